# baseline (speedup 1.0000x reference)
import functools

import jax
import jax.numpy as jnp
from jax import lax
from jax.experimental import pallas as pl
from jax.experimental.pallas import tpu as pltpu
from jax.experimental.pallas import tpu_sc as plsc


def _proj_body(t_ref, w_ref, b_ref, o_ref):
    o_ref[...] = jnp.maximum(
        jnp.dot(t_ref[...], w_ref[...], preferred_element_type=jnp.float32)
        + b_ref[...],
        0.0,
    )


def _project(table, W, b):
    V, E = table.shape
    O = W.shape[1]
    BR = 1000
    return pl.pallas_call(
        _proj_body,
        grid=(V // BR,),
        in_specs=[
            pl.BlockSpec((BR, E), lambda i: (i, 0)),
            pl.BlockSpec((E, O), lambda i: (0, 0)),
            pl.BlockSpec((1, O), lambda i: (0, 0)),
        ],
        out_specs=pl.BlockSpec((BR, O), lambda i: (i, 0)),
        out_shape=jax.ShapeDtypeStruct((V, O), jnp.float32),
    )(table, W, b.reshape(1, O))


@functools.lru_cache(maxsize=None)
def _make_gather(V, O, Bm, S, CW):
    info = plsc.get_sparse_core_info()
    NC, NS = info.num_cores, info.num_subcores
    NW = NC * NS
    assert Bm % (NW * CW) == 0
    bpw = Bm // NW
    kpt = bpw // CW
    chunks = S * kpt
    mesh = plsc.VectorSubcoreMesh(core_axis_name="c", subcore_axis_name="s")

    NB = 4
    assert chunks % NB == 0

    @functools.partial(
        pl.kernel,
        mesh=mesh,
        out_type=jax.ShapeDtypeStruct((S, Bm, O), jnp.float32),
        scratch_types=[
            pltpu.VMEM((S, bpw), jnp.int32),
            pltpu.VMEM((NB, CW, O), jnp.float32),
            [pltpu.SemaphoreType.DMA] * NB,
            [pltpu.SemaphoreType.DMA] * NB,
        ],
    )
    def gather(tbl_hbm, idx_hbm, out_hbm, idx_v, rows_v, gsems, ssems):
        wid = lax.axis_index("s") * NC + lax.axis_index("c")
        col0 = wid * bpw
        pltpu.sync_copy(idx_hbm.at[:, pl.ds(col0, bpw)], idx_v)

        def chunk_idx(c):
            t = c // kpt
            k = lax.rem(c, kpt)
            return idx_v.at[t, pl.ds(k * CW, CW)]

        def out_slab(c):
            t = c // kpt
            k = lax.rem(c, kpt)
            return out_hbm.at[t].at[pl.ds(col0 + k * CW, CW)]

        for b in range(2):
            pltpu.async_copy(tbl_hbm.at[chunk_idx(b)], rows_v.at[b], gsems[b])

        def step(i, carry):
            c0 = i * NB
            for b in range(NB):
                c = c0 + b
                pltpu.make_async_copy(
                    tbl_hbm.at[chunk_idx(c)], rows_v.at[b], gsems[b]
                ).wait()
                pltpu.async_copy(rows_v.at[b], out_slab(c), ssems[b])
                b2 = (b + 2) % NB

                @pl.when(c + 2 < chunks)
                def _():
                    @pl.when(c >= 2)
                    def _():
                        pltpu.make_async_copy(
                            rows_v.at[b2], out_slab(c), ssems[b2]
                        ).wait()

                    pltpu.async_copy(
                        tbl_hbm.at[chunk_idx(c + 2)], rows_v.at[b2], gsems[b2]
                    )

            return carry

        lax.fori_loop(0, chunks // NB, step, 0)

        for b in range(NB):
            pltpu.make_async_copy(rows_v.at[b], out_slab(0), ssems[b]).wait()

    return gather


def kernel(input_ids, table, W, b):
    Bm, S = input_ids.shape
    V, E = table.shape
    O = W.shape[1]
    CW = 32
    proj = _project(table, W, b)
    ids_t = jnp.transpose(input_ids.astype(jnp.int32))
    out_t = _make_gather(V, O, Bm, S, CW)(proj, ids_t)
    return jnp.transpose(out_t, (1, 0, 2))


# speedup vs baseline: 1.0005x; 1.0005x over previous
import functools

import jax
import jax.numpy as jnp
from jax import lax
from jax.experimental import pallas as pl
from jax.experimental.pallas import tpu as pltpu
from jax.experimental.pallas import tpu_sc as plsc


def _proj_body(t_ref, w_ref, b_ref, o_ref):
    o_ref[...] = jnp.maximum(
        jnp.dot(t_ref[...], w_ref[...], preferred_element_type=jnp.float32)
        + b_ref[...],
        0.0,
    )


def _project(table, W, b):
    V, E = table.shape
    O = W.shape[1]
    BR = 1000
    return pl.pallas_call(
        _proj_body,
        grid=(V // BR,),
        in_specs=[
            pl.BlockSpec((BR, E), lambda i: (i, 0)),
            pl.BlockSpec((E, O), lambda i: (0, 0)),
            pl.BlockSpec((1, O), lambda i: (0, 0)),
        ],
        out_specs=pl.BlockSpec((BR, O), lambda i: (i, 0)),
        out_shape=jax.ShapeDtypeStruct((V, O), jnp.float32),
    )(table, W, b.reshape(1, O))


@functools.lru_cache(maxsize=None)
def _make_gather(V, O, Bm, S, CW):
    info = plsc.get_sparse_core_info()
    NC, NS = info.num_cores, info.num_subcores
    NW = NC * NS
    assert Bm % (NW * CW) == 0
    bpw = Bm // NW
    kpt = bpw // CW
    chunks = S * kpt
    mesh = plsc.VectorSubcoreMesh(core_axis_name="c", subcore_axis_name="s")

    @functools.partial(
        pl.kernel,
        mesh=mesh,
        out_type=jax.ShapeDtypeStruct((S, Bm, O), jnp.float32),
        scratch_types=[
            pltpu.VMEM((S, bpw), jnp.int32),
            pltpu.VMEM((2, CW, O), jnp.float32),
            pltpu.SemaphoreType.DMA,
            pltpu.SemaphoreType.DMA,
        ],
    )
    def gather(tbl_hbm, idx_hbm, out_hbm, idx_v, rows_v, sem0, sem1):
        wid = lax.axis_index("s") * NC + lax.axis_index("c")
        sems = (sem0, sem1)
        col0 = wid * bpw
        pltpu.sync_copy(idx_hbm.at[:, pl.ds(col0, bpw)], idx_v)

        def chunk_idx(c):
            t = c // kpt
            k = lax.rem(c, kpt)
            return idx_v.at[t, pl.ds(k * CW, CW)]

        for b in range(2):
            pltpu.async_copy(tbl_hbm.at[chunk_idx(b)], rows_v.at[b], sems[b])

        def step(i, carry):
            c0 = i * 2
            for b in range(2):
                c = c0 + b
                t = c // kpt
                k = lax.rem(c, kpt)
                pltpu.make_async_copy(
                    tbl_hbm.at[chunk_idx(c)], rows_v.at[b], sems[b]
                ).wait()
                pltpu.sync_copy(
                    rows_v.at[b],
                    out_hbm.at[t].at[pl.ds(col0 + k * CW, CW)],
                )

                @pl.when(c + 2 < chunks)
                def _():
                    pltpu.async_copy(
                        tbl_hbm.at[chunk_idx(c + 2)], rows_v.at[b], sems[b]
                    )

            return carry

        lax.fori_loop(0, chunks // 2, step, 0)

    return gather


def kernel(input_ids, table, W, b):
    Bm, S = input_ids.shape
    V, E = table.shape
    O = W.shape[1]
    CW = 64
    proj = _project(table, W, b)
    ids_t = jnp.transpose(input_ids.astype(jnp.int32))
    out_t = _make_gather(V, O, Bm, S, CW)(proj, ids_t)
    return jnp.transpose(out_t, (1, 0, 2))
